# baseline (speedup 1.0000x reference)
import numpy as np
import jax
import jax.numpy as jnp
from jax import lax
from jax.experimental import pallas as pl
from jax.experimental.pallas import tpu as pltpu
from jax.experimental.pallas import tpu_sc as plsc

_D = 64
_S = 20
_M = 50
_B = 1024

_NC, _NS = 2, 16
_NW = _NC * _NS
_ROWS = _B * _M
_RPW = _ROWS // _NW
_CH = 80
_NCH = _RPW // _CH
_G = _CH * _S
_GPART = [(j * 128, 128) for j in range(12)] + [(1536, 64)]

_DEINT = np.arange(_D).reshape(2, 16, 2).transpose(0, 2, 1).reshape(-1)


def _pos_enc_deint():
    j = np.arange(1, _S + 1, dtype=np.float32)[:, None]
    k = np.arange(1, _D + 1, dtype=np.float32)[None, :]
    enc = (1.0 - j / _S) - (k / _D) * (1.0 - 2.0 * j / _S)
    return jnp.asarray(enc[:, _DEINT])


def _body(x_hbm, table_hbm, te_hbm, w_hbm, out_hbm,
          idx0, idx1, rows0, rows1, out_v, te_v, w_v, sem0, sem1):
    wid = lax.axis_index("s") * _NC + lax.axis_index("c")
    base = wid * _RPW
    pltpu.sync_copy(w_hbm, w_v)
    pltpu.sync_copy(te_hbm, te_v)
    idx_b, rows_b, sem_b = (idx0, idx1), (rows0, rows1), (sem0, sem1)

    def fire(c, b):
        pltpu.sync_copy(x_hbm.at[pl.ds((base + c * _CH) * _S, _G)], idx_b[b])
        for off, sz in _GPART:
            pltpu.make_async_copy(
                table_hbm.at[idx_b[b].at[pl.ds(off, sz)]],
                rows_b[b].at[pl.ds(off, sz)],
                sem_b[b],
            ).start()

    def drain(b):
        pltpu.make_async_copy(
            table_hbm.at[pl.ds(0, _G)], rows_b[b], sem_b[b]
        ).wait()

    def compute(c, b):
        rows_v = rows_b[b]
        m0 = lax.rem(c * _CH, _M)

        def row(r, carry2):
            r0 = r * _S
            mr = lax.rem(m0 + r, _M)
            for jj in range(_D // 32):
                la, lb = pl.ds(jj * 32, 16), pl.ds(jj * 32 + 16, 16)
                acc_a = te_v[mr, la]
                acc_b = te_v[mr, lb]
                for k in range(_S):
                    ab = plsc.bitcast(
                        rows_v[r0 + k, pl.ds(jj * 16, 16)], jnp.bfloat16
                    )
                    a, b2 = plsc.unpack(ab, format=plsc.PackFormat.INTERLEAVED)
                    acc_a = acc_a + a * w_v[k, la]
                    acc_b = acc_b + b2 * w_v[k, lb]
                out_v[r, la] = acc_a
                out_v[r, lb] = acc_b
            return carry2

        lax.fori_loop(0, _CH, row, 0)
        pltpu.sync_copy(out_v, out_hbm.at[pl.ds(base + c * _CH, _CH)])

    fire(0, 0)
    fire(1, 1)

    def pair(i, carry):
        for b in range(2):
            c = i * 2 + b
            drain(b)
            compute(c, b)

            @pl.when(c + 2 < _NCH)
            def _():
                fire(c + 2, b)
        return carry

    lax.fori_loop(0, _NCH // 2, pair, 0)


def kernel(x, embedding, temporal_embedding):
    table = lax.bitcast_convert_type(
        embedding.astype(jnp.bfloat16).reshape(100000, _D // 2, 2), jnp.int32
    )
    te = temporal_embedding[:, jnp.asarray(_DEINT)]
    mesh = plsc.VectorSubcoreMesh(core_axis_name="c", subcore_axis_name="s")
    out = pl.kernel(
        _body,
        mesh=mesh,
        compiler_params=pltpu.CompilerParams(
            use_tc_tiling_on_sc=False, needs_layout_passes=False
        ),
        out_type=jax.ShapeDtypeStruct((_ROWS, _D), jnp.float32),
        scratch_types=[
            pltpu.VMEM((_G,), jnp.int32),
            pltpu.VMEM((_G,), jnp.int32),
            pltpu.VMEM((_G, _D // 2), jnp.int32),
            pltpu.VMEM((_G, _D // 2), jnp.int32),
            pltpu.VMEM((_CH, _D), jnp.float32),
            pltpu.VMEM((_M, _D), jnp.float32),
            pltpu.VMEM((_S, _D), jnp.float32),
            pltpu.SemaphoreType.DMA,
            pltpu.SemaphoreType.DMA,
        ],
    )(x.reshape(-1), table, te, _pos_enc_deint())
    return (
        out.reshape(_ROWS, 2, 2, 16)
        .transpose(0, 1, 3, 2)
        .reshape(_B, _M, _D)
    )


# speedup vs baseline: 1.7394x; 1.7394x over previous
import numpy as np
import jax
import jax.numpy as jnp
from jax import lax
from jax.experimental import pallas as pl
from jax.experimental.pallas import tpu as pltpu
from jax.experimental.pallas import tpu_sc as plsc

_D = 64
_S = 20
_M = 50
_B = 1024

_NC, _NS = 2, 16
_NW = _NC * _NS
_ROWS = _B * _M
_RPW = _ROWS // _NW
_CH = 80
_NCH = _RPW // _CH
_G = _CH * _S
_GPART = [(j * 128, 128) for j in range(12)] + [(1536, 64)]

_DEINT = np.arange(_D).reshape(2, 16, 2).transpose(0, 2, 1).reshape(-1)


def _pos_enc_deint():
    j = np.arange(1, _S + 1, dtype=np.float32)[:, None]
    k = np.arange(1, _D + 1, dtype=np.float32)[None, :]
    enc = (1.0 - j / _S) - (k / _D) * (1.0 - 2.0 * j / _S)
    return jnp.asarray(enc[:, _DEINT])


def _body(x_hbm, table_hbm, te_hbm, w_hbm, out_hbm,
          idx0, idx1, rows0, rows1, out_v, te_v, w_v, sem0, sem1):
    wid = lax.axis_index("s") * _NC + lax.axis_index("c")
    base = wid * _RPW
    pltpu.sync_copy(w_hbm, w_v)
    pltpu.sync_copy(te_hbm, te_v)
    idx_b, rows_b, sem_b = (idx0, idx1), (rows0, rows1), (sem0, sem1)

    def fire(c, b):
        pltpu.sync_copy(x_hbm.at[pl.ds((base + c * _CH) * _S, _G)], idx_b[b])
        for off, sz in _GPART:
            pltpu.make_async_copy(
                table_hbm.at[idx_b[b].at[pl.ds(off, sz)]],
                rows_b[b].at[pl.ds(off, sz)],
                sem_b[b],
            ).start()

    def drain(b):
        pltpu.make_async_copy(
            table_hbm.at[pl.ds(0, _G)], rows_b[b], sem_b[b]
        ).wait()

    def compute(c, b):
        rows_v = rows_b[b]
        m0 = lax.rem(c * _CH, _M)

        def row(r, carry2):
            r0 = r * _S
            mr = lax.rem(m0 + r, _M)
            rsplat = jnp.full((16,), r, jnp.int32)
            for jj in range(_D // 32):
                la, lb = pl.ds(jj * 32, 16), pl.ds(jj * 32 + 16, 16)
                acc_a = te_v[mr, la]
                acc_b = te_v[mr, lb]
                for k in range(_S):
                    ab = rows_v[r0 + k, pl.ds(jj * 32, 32)]
                    a, b2 = plsc.unpack(ab, format=plsc.PackFormat.INTERLEAVED)
                    acc_a = acc_a + a * w_v[k, la]
                    acc_b = acc_b + b2 * w_v[k, lb]
                even = lax.iota(jnp.int32, 16) * 2 + jj * 32
                plsc.store_scatter(out_v, [rsplat, even], acc_a)
                plsc.store_scatter(out_v, [rsplat, even + 1], acc_b)
            return carry2

        lax.fori_loop(0, _CH, row, 0)
        pltpu.sync_copy(out_v, out_hbm.at[pl.ds(base + c * _CH, _CH)])

    fire(0, 0)
    fire(1, 1)

    def pair(i, carry):
        for b in range(2):
            c = i * 2 + b
            drain(b)
            compute(c, b)

            @pl.when(c + 2 < _NCH)
            def _():
                fire(c + 2, b)
        return carry

    lax.fori_loop(0, _NCH // 2, pair, 0)


def kernel(x, embedding, temporal_embedding):
    table = embedding.astype(jnp.bfloat16)
    te = temporal_embedding[:, jnp.asarray(_DEINT)]
    mesh = plsc.VectorSubcoreMesh(core_axis_name="c", subcore_axis_name="s")
    out = pl.kernel(
        _body,
        mesh=mesh,
        compiler_params=pltpu.CompilerParams(
            use_tc_tiling_on_sc=False, needs_layout_passes=False
        ),
        out_type=jax.ShapeDtypeStruct((_ROWS, _D), jnp.float32),
        scratch_types=[
            pltpu.VMEM((_G,), jnp.int32),
            pltpu.VMEM((_G,), jnp.int32),
            pltpu.VMEM((_G, _D), jnp.bfloat16),
            pltpu.VMEM((_G, _D), jnp.bfloat16),
            pltpu.VMEM((_CH, _D), jnp.float32),
            pltpu.VMEM((_M, _D), jnp.float32),
            pltpu.VMEM((_S, _D), jnp.float32),
            pltpu.SemaphoreType.DMA,
            pltpu.SemaphoreType.DMA,
        ],
    )(x.reshape(-1), table, te, _pos_enc_deint())
    return out.reshape(_B, _M, _D)


# speedup vs baseline: 1.8310x; 1.0527x over previous
import numpy as np
import jax
import jax.numpy as jnp
from jax import lax
from jax.experimental import pallas as pl
from jax.experimental.pallas import tpu as pltpu
from jax.experimental.pallas import tpu_sc as plsc

_D = 64
_S = 20
_M = 50
_B = 1024

_NC, _NS = 2, 16
_NW = _NC * _NS
_ROWS = _B * _M
_RPW = _ROWS // _NW
_CH = 80
_NCH = _RPW // _CH
_G = _CH * _S
_GPART = [(j * 128, 128) for j in range(12)] + [(1536, 64)]

_DEINT = np.arange(_D).reshape(2, 16, 2).transpose(0, 2, 1).reshape(-1)


def _pos_enc_deint():
    j = np.arange(1, _S + 1, dtype=np.float32)[:, None]
    k = np.arange(1, _D + 1, dtype=np.float32)[None, :]
    enc = (1.0 - j / _S) - (k / _D) * (1.0 - 2.0 * j / _S)
    return jnp.asarray(enc[:, _DEINT])


def _body(x_hbm, table_hbm, te_hbm, w_hbm, out_hbm,
          idx0, idx1, rows0, rows1, out0, out1, te_v, w_v,
          sem0, sem1, semi0, semi1, semo0, semo1):
    wid = lax.axis_index("s") * _NC + lax.axis_index("c")
    base = wid * _RPW
    pltpu.sync_copy(w_hbm, w_v)
    pltpu.sync_copy(te_hbm, te_v)
    idx_b, rows_b, sem_b = (idx0, idx1), (rows0, rows1), (sem0, sem1)
    out_b, semi_b, semo_b = (out0, out1), (semi0, semi1), (semo0, semo1)

    def fire_idx(c, b):
        pltpu.make_async_copy(
            x_hbm.at[pl.ds((base + c * _CH) * _S, _G)], idx_b[b], semi_b[b]
        ).start()

    def fire_gathers(c, b):
        pltpu.make_async_copy(
            x_hbm.at[pl.ds(0, _G)], idx_b[b], semi_b[b]
        ).wait()
        for off, sz in _GPART:
            pltpu.make_async_copy(
                table_hbm.at[idx_b[b].at[pl.ds(off, sz)]],
                rows_b[b].at[pl.ds(off, sz)],
                sem_b[b],
            ).start()

    def drain(b):
        pltpu.make_async_copy(
            table_hbm.at[pl.ds(0, _G)], rows_b[b], sem_b[b]
        ).wait()

    def compute(c, b):
        rows_v = rows_b[b]
        out_v = out_b[b]
        m0 = lax.rem(c * _CH, _M)

        @pl.when(c >= 2)
        def _():
            pltpu.make_async_copy(
                out_v, out_hbm.at[pl.ds(0, _CH)], semo_b[b]
            ).wait()

        def row(r, carry2):
            r0 = r * _S
            mr = lax.rem(m0 + r, _M)
            rsplat = jnp.full((16,), r, jnp.int32)
            for jj in range(_D // 32):
                la, lb = pl.ds(jj * 32, 16), pl.ds(jj * 32 + 16, 16)
                acc_a = te_v[mr, la]
                acc_b = te_v[mr, lb]
                for k in range(_S):
                    ab = rows_v[r0 + k, pl.ds(jj * 32, 32)]
                    a, b2 = plsc.unpack(ab, format=plsc.PackFormat.INTERLEAVED)
                    acc_a = acc_a + a * w_v[k, la]
                    acc_b = acc_b + b2 * w_v[k, lb]
                even = lax.iota(jnp.int32, 16) * 2 + jj * 32
                plsc.store_scatter(out_v, [rsplat, even], acc_a)
                plsc.store_scatter(out_v, [rsplat, even + 1], acc_b)
            return carry2

        lax.fori_loop(0, _CH, row, 0)
        pltpu.make_async_copy(
            out_v, out_hbm.at[pl.ds(base + c * _CH, _CH)], semo_b[b]
        ).start()

    fire_idx(0, 0)
    fire_idx(1, 1)
    fire_gathers(0, 0)
    fire_gathers(1, 1)

    def pair(i, carry):
        for b in range(2):
            c = i * 2 + b

            drain(b)

            @pl.when(c + 2 < _NCH)
            def _():
                fire_idx(c + 2, b)

            compute(c, b)

            @pl.when(c + 2 < _NCH)
            def _():
                fire_gathers(c + 2, b)
        return carry

    lax.fori_loop(0, _NCH // 2, pair, 0)
    pltpu.make_async_copy(out0, out_hbm.at[pl.ds(0, _CH)], semo0).wait()
    pltpu.make_async_copy(out1, out_hbm.at[pl.ds(0, _CH)], semo1).wait()


def kernel(x, embedding, temporal_embedding):
    table = embedding.astype(jnp.bfloat16)
    te = temporal_embedding[:, jnp.asarray(_DEINT)]
    mesh = plsc.VectorSubcoreMesh(core_axis_name="c", subcore_axis_name="s")
    out = pl.kernel(
        _body,
        mesh=mesh,
        compiler_params=pltpu.CompilerParams(
            use_tc_tiling_on_sc=False, needs_layout_passes=False
        ),
        out_type=jax.ShapeDtypeStruct((_ROWS, _D), jnp.float32),
        scratch_types=[
            pltpu.VMEM((_G,), jnp.int32),
            pltpu.VMEM((_G,), jnp.int32),
            pltpu.VMEM((_G, _D), jnp.bfloat16),
            pltpu.VMEM((_G, _D), jnp.bfloat16),
            pltpu.VMEM((_CH, _D), jnp.float32),
            pltpu.VMEM((_CH, _D), jnp.float32),
            pltpu.VMEM((_M, _D), jnp.float32),
            pltpu.VMEM((_S, _D), jnp.float32),
            pltpu.SemaphoreType.DMA,
            pltpu.SemaphoreType.DMA,
            pltpu.SemaphoreType.DMA,
            pltpu.SemaphoreType.DMA,
            pltpu.SemaphoreType.DMA,
            pltpu.SemaphoreType.DMA,
        ],
    )(x.reshape(-1), table, te, _pos_enc_deint())
    return out.reshape(_B, _M, _D)
